# baseline (speedup 1.0000x reference)
import functools

import jax
import jax.numpy as jnp
from jax import lax
from jax.experimental import pallas as pl
from jax.experimental.pallas import tpu as pltpu
from jax.experimental.pallas import tpu_sc as plsc

B = 2048
NPG = 39
M = 40
DEG = 16
EPG = NPG * DEG
GM2 = M * M
NW = 32
GPW = B // NW
GPR = 16
ROUNDS = GPW // GPR

G = 32
ROWS = G * M
STEPS = B // G

MLP_ROWS = 256
F_DIM = 4560


def _make_sc_body(nb, hoff):
    gpw = nb // NW

    def _sc_body(src_hbm, dst_hbm, out_hbm, acc, sbuf, dbuf):
        c = lax.axis_index("c")
        s = lax.axis_index("s")
        wid = s * 2 + c
        lane = lax.iota(jnp.int32, 16)
        ones = jnp.ones((16,), jnp.float32)
        zeros = jnp.zeros((16,), jnp.float32)

        def round_body(r, carry):
            gl = wid * gpw + r * GPR

            def zero_body(i, carry2):
                for u in range(8):
                    acc[pl.ds((i * 8 + u) * 16, 16)] = zeros
                return carry2

            lax.fori_loop(0, GPR * GM2 // 128, zero_body, 0)

            pltpu.sync_copy(src_hbm.at[pl.ds(gl * EPG, GPR * EPG)], sbuf)
            pltpu.sync_copy(dst_hbm.at[pl.ds(gl * EPG, GPR * EPG)], dbuf)

            gbase = (hoff + gl + lane) * NPG
            abase = lane * GM2

            def edge_body(j, carry2):
                for u in range(8):
                    ei = lane * EPG + (j * 8 + u)
                    sv = plsc.load_gather(sbuf, [ei])
                    dv = plsc.load_gather(dbuf, [ei])
                    idx = abase + (dv - gbase) * M + (sv - gbase)
                    plsc.addupdate_scatter(acc, [idx], ones)
                return carry2

            lax.fori_loop(0, EPG // 8, edge_body, 0)

            pltpu.sync_copy(acc, out_hbm.at[pl.ds(gl * GM2, GPR * GM2)])
            return carry

        lax.fori_loop(0, gpw // GPR, round_body, 0)

    return _sc_body


def _build_a(src, dst, nb, hoff):
    mesh = plsc.VectorSubcoreMesh(core_axis_name="c", subcore_axis_name="s")
    call = functools.partial(
        pl.kernel,
        mesh=mesh,
        compiler_params=pltpu.CompilerParams(needs_layout_passes=False),
        out_type=jax.ShapeDtypeStruct((nb * GM2,), jnp.float32),
        scratch_types=[
            pltpu.VMEM((GPR * GM2,), jnp.float32),
            pltpu.VMEM((GPR * EPG,), jnp.int32),
            pltpu.VMEM((GPR * EPG,), jnp.int32),
        ],
    )(_make_sc_body(nb, hoff))
    return call(src, dst)


def _gat_block_kernel(x_ref, a_ref,
                      w1, b1, w2, b2, w3, b3, w4, b4,
                      h1_ref, h2_ref, h3_ref, h4_ref, pcat_ref):
    f32 = jnp.float32
    rowl = lax.broadcasted_iota(jnp.int32, (ROWS, 1), 0) % M
    valid = rowl < NPG
    lane_id = lax.broadcasted_iota(jnp.int32, (ROWS, M), 1)
    patt = (lane_id == rowl).astype(f32)
    self_loop = jnp.where((lane_id == rowl) & valid, 1.0, 0.0)
    acnt = a_ref[...] + self_loop
    mask = acnt > 0

    sm = (lax.broadcasted_iota(jnp.int32, (G, ROWS), 1) // M
          == lax.broadcasted_iota(jnp.int32, (G, ROWS), 0)).astype(f32)
    rm = (lax.broadcasted_iota(jnp.int32, (ROWS, G), 0) // M
          == lax.broadcasted_iota(jnp.int32, (ROWS, G), 1)).astype(f32)
    ones_col = jnp.ones((ROWS, 1), f32)
    ones_row = jnp.ones((1, M), f32)

    def gat_layer(h_in, waug, b_row, fout, first):
        if first:
            h2aug = h_in * waug[...]
        else:
            h2aug = jnp.dot(h_in, waug[...], preferred_element_type=f32)
        h2 = h2aug[:, :fout]
        asrc = h2aug[:, fout:fout + 1]
        adst = h2aug[:, fout + 1:fout + 2]
        rhs = asrc * patt
        grid = jnp.dot(sm, rhs, preferred_element_type=f32)
        r_mat = jnp.dot(rm, grid, preferred_element_type=f32)
        pre = r_mat + adst
        alpha = jnp.where(pre >= 0, pre, 0.2 * pre)
        am = jnp.max(alpha, axis=1, keepdims=True)
        shift = jnp.maximum(am, 0.0)
        p_mat = jnp.where(mask, acnt * jnp.exp(alpha - shift), 0.0)
        h2e = jnp.concatenate([h2, ones_col], axis=1)
        num_aug = jnp.concatenate(
            [jnp.dot(p_mat[t * M:(t + 1) * M, :],
                     h2e[t * M:(t + 1) * M, :],
                     preferred_element_type=f32) for t in range(G)], axis=0)
        num = num_aug[:, :fout]
        den = num_aug[:, fout:fout + 1]
        inv = 1.0 / (den + 1e-16)
        return jnp.maximum(num * inv + b_row[...], 0.0)

    def pool(h, fdim):
        hm = jnp.where(valid, h, -3e38)
        return jnp.max(hm.reshape(G, M, fdim), axis=1)

    x = x_ref[...]
    p0 = pool(x, 1)
    h1 = gat_layer(x, w1, b1, 8, True)
    h1_ref[...] = h1
    p1 = pool(h1, 8)
    h2 = gat_layer(h1, w2, b2, 64, False)
    h2_ref[...] = h2
    p2 = pool(h2, 64)
    h3 = gat_layer(h2, w3, b3, 32, False)
    h3_ref[...] = h3
    p3 = pool(h3, 32)
    h4 = gat_layer(h3, w4, b4, 9, False)
    h4_ref[...] = h4
    p4 = pool(h4, 9)
    pcat_ref[...] = jnp.concatenate([p0, p1, p2, p3, p4], axis=1)


def _run_gat(xp, a2, weights):
    (w1, b1, w2, b2, w3, b3, w4, b4) = weights
    nb = xp.shape[0] // M
    f32 = jnp.float32
    data_spec = lambda cols: pl.BlockSpec((ROWS, cols), lambda i: (i, 0))
    w_spec = lambda r, c: pl.BlockSpec((r, c), lambda i: (0, 0))
    pool_spec = lambda cols: pl.BlockSpec((G, cols), lambda i: (i, 0))
    out_shapes = [
        jax.ShapeDtypeStruct((nb * M, 8), f32),
        jax.ShapeDtypeStruct((nb * M, 64), f32),
        jax.ShapeDtypeStruct((nb * M, 32), f32),
        jax.ShapeDtypeStruct((nb * M, 9), f32),
        jax.ShapeDtypeStruct((nb, 114), f32),
    ]
    in_specs = [
        data_spec(1), data_spec(M),
        w_spec(1, 10), w_spec(1, 8),
        w_spec(8, 66), w_spec(1, 64),
        w_spec(64, 34), w_spec(1, 32),
        w_spec(32, 11), w_spec(1, 9),
    ]
    out_specs = [
        data_spec(8), data_spec(64), data_spec(32), data_spec(9),
        pool_spec(114),
    ]
    return pl.pallas_call(
        _gat_block_kernel,
        grid=(nb // G,),
        in_specs=in_specs,
        out_specs=out_specs,
        out_shape=out_shapes,
        compiler_params=pltpu.CompilerParams(dimension_semantics=("parallel",)),
    )(xp, a2, w1, b1, w2, b2, w3, b3, w4, b4)


def _mlp_kernel(res_ref, r1_ref, r2_ref, r3_ref, r4_ref, pc_ref,
                wr, wr1, wr2, wr3, wr4, wpc, bl1, wl2, bl2, wl3, bl3, o_ref):
    f32 = jnp.float32
    t = (jnp.dot(res_ref[...], wr[...], preferred_element_type=f32)
         + jnp.dot(r1_ref[...], wr1[...], preferred_element_type=f32)
         + jnp.dot(r2_ref[...], wr2[...], preferred_element_type=f32)
         + jnp.dot(r3_ref[...], wr3[...], preferred_element_type=f32)
         + jnp.dot(r4_ref[...], wr4[...], preferred_element_type=f32)
         + jnp.dot(pc_ref[...], wpc[...], preferred_element_type=f32)
         + bl1[...])
    t = jnp.maximum(t, 0.0)
    t = jnp.dot(t, wl2[...], preferred_element_type=f32) + bl2[...]
    t = jnp.maximum(t, 0.0)
    o_ref[...] = jnp.dot(t, wl3[...], preferred_element_type=f32) + bl3[...]


def _run_mlp(pieces, wl1_parts, bl1, wl2, bl2, wl3, bl3):
    f32 = jnp.float32
    nb = pieces[0].shape[0]
    w_spec = lambda r, c: pl.BlockSpec((r, c), lambda i: (0, 0))
    d_spec = lambda cols: pl.BlockSpec((MLP_ROWS, cols), lambda i: (i, 0))
    piece_cols = [NPG, NPG * 8, NPG * 64, NPG * 32, NPG * 9, 114]
    return pl.pallas_call(
        _mlp_kernel,
        grid=(nb // MLP_ROWS,),
        in_specs=(
            [d_spec(c) for c in piece_cols]
            + [w_spec(c, 1024) for c in piece_cols]
            + [w_spec(1, 1024), w_spec(1024, 128), w_spec(1, 128),
               w_spec(128, 9), w_spec(1, 9)]
        ),
        out_specs=pl.BlockSpec((MLP_ROWS, 9), lambda i: (i, 0)),
        out_shape=jax.ShapeDtypeStruct((nb, 9), f32),
        compiler_params=pltpu.CompilerParams(dimension_semantics=("parallel",)),
    )(*pieces, *wl1_parts, bl1, wl2, bl2, wl3, bl3)


def kernel(x, edge_index, batch, W1, as1, ad1, b1, W2, as2, ad2, b2,
           W3, as3, ad3, b3, W4, as4, ad4, b4, Wl1, bl1, Wl2, bl2, Wl3, bl3):
    del batch
    f32 = jnp.float32
    src = edge_index[0]
    dst = edge_index[1]

    row = lambda v: v.reshape(1, -1).astype(f32)
    aug = lambda w, a_s, a_d: jnp.concatenate(
        [w, (w @ a_s)[:, None], (w @ a_d)[:, None]], axis=1)
    weights = (aug(W1, as1, ad1), row(b1), aug(W2, as2, ad2), row(b2),
               aug(W3, as3, ad3), row(b3), aug(W4, as4, ad4), row(b4))
    offs = [0, 39, 351, 2847, 4095, 4446, 4560]
    wl1_parts = [Wl1[offs[i]:offs[i + 1]] for i in range(6)]
    bl = (row(bl1), Wl2, row(bl2), Wl3, row(bl3))

    nh = B // 2
    outs = []
    for h in range(2):
        srch = src[h * nh * EPG:(h + 1) * nh * EPG]
        dsth = dst[h * nh * EPG:(h + 1) * nh * EPG]
        xh = x[h * nh * NPG:(h + 1) * nh * NPG]

        a2 = _build_a(srch, dsth, nh, h * nh).reshape(nh * M, M)
        xph = jnp.pad(xh.reshape(nh, NPG),
                      ((0, 0), (0, M - NPG))).reshape(nh * M, 1)
        h1, h2, h3, h4, pcat = _run_gat(xph, a2, weights)

        res = xh.reshape(nh, NPG)
        res1 = h1.reshape(nh, M, 8)[:, :NPG, :].reshape(nh, NPG * 8)
        res2 = h2.reshape(nh, M, 64)[:, :NPG, :].reshape(nh, NPG * 64)
        res3 = h3.reshape(nh, M, 32)[:, :NPG, :].reshape(nh, NPG * 32)
        res4 = h4.reshape(nh, M, 9)[:, :NPG, :].reshape(nh, NPG * 9)
        pieces = (res, res1, res2, res3, res4, pcat)
        outs.append(_run_mlp(pieces, wl1_parts, *bl))

    return jnp.concatenate(outs, axis=0)


# speedup vs baseline: 1.0348x; 1.0348x over previous
import functools

import jax
import jax.numpy as jnp
from jax import lax
from jax.experimental import pallas as pl
from jax.experimental.pallas import tpu as pltpu
from jax.experimental.pallas import tpu_sc as plsc

B = 2048
NPG = 39
M = 40
DEG = 16
EPG = NPG * DEG
GM2 = M * M
NW = 32
GPW = B // NW
GPR = 16
ROUNDS = GPW // GPR

G = 32
ROWS = G * M
STEPS = B // G

MLP_ROWS = 256
F_DIM = 4560


def _make_sc_body(nb, hoff):
    gpw = nb // NW

    def _sc_body(src_hbm, dst_hbm, out_hbm, acc, sbuf, dbuf):
        c = lax.axis_index("c")
        s = lax.axis_index("s")
        wid = s * 2 + c
        lane = lax.iota(jnp.int32, 16)
        ones = jnp.ones((16,), jnp.float32)
        zeros = jnp.zeros((16,), jnp.float32)

        def round_body(r, carry):
            gl = wid * gpw + r * GPR

            def zero_body(i, carry2):
                for u in range(8):
                    acc[pl.ds((i * 8 + u) * 16, 16)] = zeros
                return carry2

            lax.fori_loop(0, GPR * GM2 // 128, zero_body, 0)

            pltpu.sync_copy(src_hbm.at[pl.ds(gl * EPG, GPR * EPG)], sbuf)
            pltpu.sync_copy(dst_hbm.at[pl.ds(gl * EPG, GPR * EPG)], dbuf)

            gbase = (hoff + gl + lane) * NPG
            abase = lane * GM2

            def edge_body(j, carry2):
                for u in range(8):
                    ei = lane * EPG + (j * 8 + u)
                    sv = plsc.load_gather(sbuf, [ei])
                    dv = plsc.load_gather(dbuf, [ei])
                    idx = abase + (dv - gbase) * M + (sv - gbase)
                    plsc.addupdate_scatter(acc, [idx], ones)
                return carry2

            lax.fori_loop(0, EPG // 8, edge_body, 0)

            pltpu.sync_copy(acc, out_hbm.at[pl.ds(gl * GM2, GPR * GM2)])
            return carry

        lax.fori_loop(0, gpw // GPR, round_body, 0)

    return _sc_body


def _build_a(src, dst, nb, hoff):
    mesh = plsc.VectorSubcoreMesh(core_axis_name="c", subcore_axis_name="s")
    call = functools.partial(
        pl.kernel,
        mesh=mesh,
        compiler_params=pltpu.CompilerParams(needs_layout_passes=False),
        out_type=jax.ShapeDtypeStruct((nb * GM2,), jnp.float32),
        scratch_types=[
            pltpu.VMEM((GPR * GM2,), jnp.float32),
            pltpu.VMEM((GPR * EPG,), jnp.int32),
            pltpu.VMEM((GPR * EPG,), jnp.int32),
        ],
    )(_make_sc_body(nb, hoff))
    return call(src, dst)


def _gat_block_kernel(x_ref, a_ref,
                      w1, b1, w2, b2, w3, b3, w4, b4,
                      h1_ref, h2_ref, h3_ref, h4_ref, pcat_ref):
    f32 = jnp.float32
    rowl = lax.broadcasted_iota(jnp.int32, (ROWS, 1), 0) % M
    valid = rowl < NPG
    lane_id = lax.broadcasted_iota(jnp.int32, (ROWS, M), 1)
    patt = (lane_id == rowl).astype(f32)
    self_loop = jnp.where((lane_id == rowl) & valid, 1.0, 0.0)
    acnt = a_ref[...] + self_loop
    mask = acnt > 0

    sm = (lax.broadcasted_iota(jnp.int32, (G, ROWS), 1) // M
          == lax.broadcasted_iota(jnp.int32, (G, ROWS), 0)).astype(f32)
    rm = (lax.broadcasted_iota(jnp.int32, (ROWS, G), 0) // M
          == lax.broadcasted_iota(jnp.int32, (ROWS, G), 1)).astype(f32)
    ones_col = jnp.ones((ROWS, 1), f32)
    ones_row = jnp.ones((1, M), f32)

    def gat_layer(h_in, waug, b_row, fout, first):
        if first:
            h2aug = h_in * waug[...]
        else:
            h2aug = jnp.dot(h_in, waug[...], preferred_element_type=f32)
        h2 = h2aug[:, :fout]
        asrc = h2aug[:, fout:fout + 1]
        adst = h2aug[:, fout + 1:fout + 2]
        rhs = asrc * patt
        grid = jnp.dot(sm, rhs, preferred_element_type=f32)
        r_mat = jnp.dot(rm, grid, preferred_element_type=f32)
        pre = r_mat + adst
        alpha = jnp.where(pre >= 0, pre, 0.2 * pre)
        am = jnp.max(alpha, axis=1, keepdims=True)
        shift = jnp.maximum(am, 0.0)
        p_mat = jnp.where(mask, acnt * jnp.exp(alpha - shift), 0.0)
        h2e = jnp.concatenate([h2, ones_col], axis=1)
        num_aug = jnp.concatenate(
            [jnp.dot(p_mat[t * M:(t + 1) * M, :],
                     h2e[t * M:(t + 1) * M, :],
                     preferred_element_type=f32) for t in range(G)], axis=0)
        num = num_aug[:, :fout]
        den = num_aug[:, fout:fout + 1]
        inv = 1.0 / (den + 1e-16)
        return jnp.maximum(num * inv + b_row[...], 0.0)

    def pool(h, fdim):
        hm = jnp.where(valid, h, -3e38)
        return jnp.max(hm.reshape(G, M, fdim), axis=1)

    x = x_ref[...]
    p0 = pool(x, 1)
    h1 = gat_layer(x, w1, b1, 8, True)
    h1_ref[...] = h1
    p1 = pool(h1, 8)
    h2 = gat_layer(h1, w2, b2, 64, False)
    h2_ref[...] = h2
    p2 = pool(h2, 64)
    h3 = gat_layer(h2, w3, b3, 32, False)
    h3_ref[...] = h3
    p3 = pool(h3, 32)
    h4 = gat_layer(h3, w4, b4, 9, False)
    h4_ref[...] = h4
    p4 = pool(h4, 9)
    pcat_ref[...] = jnp.concatenate([p0, p1, p2, p3, p4], axis=1)


def _run_gat(xp, a2, weights):
    (w1, b1, w2, b2, w3, b3, w4, b4) = weights
    nb = xp.shape[0] // M
    f32 = jnp.float32
    data_spec = lambda cols: pl.BlockSpec((ROWS, cols), lambda i: (i, 0))
    w_spec = lambda r, c: pl.BlockSpec((r, c), lambda i: (0, 0))
    pool_spec = lambda cols: pl.BlockSpec((G, cols), lambda i: (i, 0))
    out_shapes = [
        jax.ShapeDtypeStruct((nb * M, 8), f32),
        jax.ShapeDtypeStruct((nb * M, 64), f32),
        jax.ShapeDtypeStruct((nb * M, 32), f32),
        jax.ShapeDtypeStruct((nb * M, 9), f32),
        jax.ShapeDtypeStruct((nb, 114), f32),
    ]
    in_specs = [
        data_spec(1), data_spec(M),
        w_spec(1, 10), w_spec(1, 8),
        w_spec(8, 66), w_spec(1, 64),
        w_spec(64, 34), w_spec(1, 32),
        w_spec(32, 11), w_spec(1, 9),
    ]
    out_specs = [
        data_spec(8), data_spec(64), data_spec(32), data_spec(9),
        pool_spec(114),
    ]
    return pl.pallas_call(
        _gat_block_kernel,
        grid=(nb // G,),
        in_specs=in_specs,
        out_specs=out_specs,
        out_shape=out_shapes,
        compiler_params=pltpu.CompilerParams(dimension_semantics=("parallel",)),
    )(xp, a2, w1, b1, w2, b2, w3, b3, w4, b4)


def _mlp_kernel(res_ref, r1_ref, r2_ref, r3_ref, r4_ref, pc_ref,
                wr, wr1, wr2, wr3, wr4, wpc, bl1, wl2, bl2, wl3, bl3, o_ref):
    f32 = jnp.float32
    t = (jnp.dot(res_ref[...], wr[...], preferred_element_type=f32)
         + jnp.dot(r1_ref[...], wr1[...], preferred_element_type=f32)
         + jnp.dot(r2_ref[...], wr2[...], preferred_element_type=f32)
         + jnp.dot(r3_ref[...], wr3[...], preferred_element_type=f32)
         + jnp.dot(r4_ref[...], wr4[...], preferred_element_type=f32)
         + jnp.dot(pc_ref[...], wpc[...], preferred_element_type=f32)
         + bl1[...])
    t = jnp.maximum(t, 0.0)
    t = jnp.dot(t, wl2[...], preferred_element_type=f32) + bl2[...]
    t = jnp.maximum(t, 0.0)
    o_ref[...] = jnp.dot(t, wl3[...], preferred_element_type=f32) + bl3[...]


def _run_mlp(pieces, wl1_parts, bl1, wl2, bl2, wl3, bl3):
    f32 = jnp.float32
    nb = pieces[0].shape[0]
    w_spec = lambda r, c: pl.BlockSpec((r, c), lambda i: (0, 0))
    d_spec = lambda cols: pl.BlockSpec((MLP_ROWS, cols), lambda i: (i, 0))
    piece_cols = [NPG, NPG * 8, NPG * 64, NPG * 32, NPG * 9, 114]
    return pl.pallas_call(
        _mlp_kernel,
        grid=(nb // MLP_ROWS,),
        in_specs=(
            [d_spec(c) for c in piece_cols]
            + [w_spec(c, 1024) for c in piece_cols]
            + [w_spec(1, 1024), w_spec(1024, 128), w_spec(1, 128),
               w_spec(128, 9), w_spec(1, 9)]
        ),
        out_specs=pl.BlockSpec((MLP_ROWS, 9), lambda i: (i, 0)),
        out_shape=jax.ShapeDtypeStruct((nb, 9), f32),
        compiler_params=pltpu.CompilerParams(dimension_semantics=("parallel",)),
    )(*pieces, *wl1_parts, bl1, wl2, bl2, wl3, bl3)


def kernel(x, edge_index, batch, W1, as1, ad1, b1, W2, as2, ad2, b2,
           W3, as3, ad3, b3, W4, as4, ad4, b4, Wl1, bl1, Wl2, bl2, Wl3, bl3):
    del batch
    f32 = jnp.float32
    src = edge_index[0]
    dst = edge_index[1]

    row = lambda v: v.reshape(1, -1).astype(f32)
    aug = lambda w, a_s, a_d: jnp.concatenate(
        [w, (w @ a_s)[:, None], (w @ a_d)[:, None]], axis=1)
    weights = (aug(W1, as1, ad1), row(b1), aug(W2, as2, ad2), row(b2),
               aug(W3, as3, ad3), row(b3), aug(W4, as4, ad4), row(b4))
    offs = [0, 39, 351, 2847, 4095, 4446, 4560]
    wl1_parts = [Wl1[offs[i]:offs[i + 1]] for i in range(6)]
    bl = (row(bl1), Wl2, row(bl2), Wl3, row(bl3))

    a2 = _build_a(src, dst, B, 0).reshape(B * M, M)
    xp = jnp.pad(x.reshape(B, NPG), ((0, 0), (0, M - NPG))).reshape(B * M, 1)
    h1, h2, h3, h4, pcat = _run_gat(xp, a2, weights)

    res = x.reshape(B, NPG)
    res1 = h1.reshape(B, M, 8)[:, :NPG, :].reshape(B, NPG * 8)
    res2 = h2.reshape(B, M, 64)[:, :NPG, :].reshape(B, NPG * 64)
    res3 = h3.reshape(B, M, 32)[:, :NPG, :].reshape(B, NPG * 32)
    res4 = h4.reshape(B, M, 9)[:, :NPG, :].reshape(B, NPG * 9)
    pieces = (res, res1, res2, res3, res4, pcat)
    return _run_mlp(pieces, wl1_parts, *bl)


# speedup vs baseline: 1.0658x; 1.0299x over previous
import functools

import jax
import jax.numpy as jnp
from jax import lax
from jax.experimental import pallas as pl
from jax.experimental.pallas import tpu as pltpu
from jax.experimental.pallas import tpu_sc as plsc

B = 2048
NPG = 39
M = 40
DEG = 16
EPG = NPG * DEG
GM2 = M * M
NW = 32
GPW = B // NW
GPR = 16
ROUNDS = GPW // GPR

G = 64
ROWS = G * M
STEPS = B // G

MLP_ROWS = 256
F_DIM = 4560


def _make_sc_body(nb, hoff):
    gpw = nb // NW

    def _sc_body(src_hbm, dst_hbm, out_hbm, acc, sbuf, dbuf):
        c = lax.axis_index("c")
        s = lax.axis_index("s")
        wid = s * 2 + c
        lane = lax.iota(jnp.int32, 16)
        ones = jnp.ones((16,), jnp.float32)
        zeros = jnp.zeros((16,), jnp.float32)

        def round_body(r, carry):
            gl = wid * gpw + r * GPR

            def zero_body(i, carry2):
                for u in range(8):
                    acc[pl.ds((i * 8 + u) * 16, 16)] = zeros
                return carry2

            lax.fori_loop(0, GPR * GM2 // 128, zero_body, 0)

            pltpu.sync_copy(src_hbm.at[pl.ds(gl * EPG, GPR * EPG)], sbuf)
            pltpu.sync_copy(dst_hbm.at[pl.ds(gl * EPG, GPR * EPG)], dbuf)

            gbase = (hoff + gl + lane) * NPG
            abase = lane * GM2

            def edge_body(j, carry2):
                for u in range(8):
                    ei = lane * EPG + (j * 8 + u)
                    sv = plsc.load_gather(sbuf, [ei])
                    dv = plsc.load_gather(dbuf, [ei])
                    idx = abase + (dv - gbase) * M + (sv - gbase)
                    plsc.addupdate_scatter(acc, [idx], ones)
                return carry2

            lax.fori_loop(0, EPG // 8, edge_body, 0)

            pltpu.sync_copy(acc, out_hbm.at[pl.ds(gl * GM2, GPR * GM2)])
            return carry

        lax.fori_loop(0, gpw // GPR, round_body, 0)

    return _sc_body


def _build_a(src, dst, nb, hoff):
    mesh = plsc.VectorSubcoreMesh(core_axis_name="c", subcore_axis_name="s")
    call = functools.partial(
        pl.kernel,
        mesh=mesh,
        compiler_params=pltpu.CompilerParams(needs_layout_passes=False),
        out_type=jax.ShapeDtypeStruct((nb * GM2,), jnp.float32),
        scratch_types=[
            pltpu.VMEM((GPR * GM2,), jnp.float32),
            pltpu.VMEM((GPR * EPG,), jnp.int32),
            pltpu.VMEM((GPR * EPG,), jnp.int32),
        ],
    )(_make_sc_body(nb, hoff))
    return call(src, dst)


def _gat_block_kernel(x_ref, a_ref,
                      w1, b1, w2, b2, w3, b3, w4, b4,
                      h1_ref, h2_ref, h3_ref, h4_ref, pcat_ref):
    f32 = jnp.float32
    rowl = lax.broadcasted_iota(jnp.int32, (ROWS, 1), 0) % M
    valid = rowl < NPG
    lane_id = lax.broadcasted_iota(jnp.int32, (ROWS, M), 1)
    patt = (lane_id == rowl).astype(f32)
    self_loop = jnp.where((lane_id == rowl) & valid, 1.0, 0.0)
    acnt = a_ref[...] + self_loop
    mask = acnt > 0

    sm = (lax.broadcasted_iota(jnp.int32, (G, ROWS), 1) // M
          == lax.broadcasted_iota(jnp.int32, (G, ROWS), 0)).astype(f32)
    rm = (lax.broadcasted_iota(jnp.int32, (ROWS, G), 0) // M
          == lax.broadcasted_iota(jnp.int32, (ROWS, G), 1)).astype(f32)
    ones_col = jnp.ones((ROWS, 1), f32)
    ones_row = jnp.ones((1, M), f32)

    def gat_layer(h_in, waug, b_row, fout, first):
        if first:
            h2aug = h_in * waug[...]
        else:
            h2aug = jnp.dot(h_in, waug[...], preferred_element_type=f32)
        h2 = h2aug[:, :fout]
        asrc = h2aug[:, fout:fout + 1]
        adst = h2aug[:, fout + 1:fout + 2]
        rhs = asrc * patt
        grid = jnp.dot(sm, rhs, preferred_element_type=f32)
        r_mat = jnp.dot(rm, grid, preferred_element_type=f32)
        pre = r_mat + adst
        alpha = jnp.where(pre >= 0, pre, 0.2 * pre)
        am = jnp.max(alpha, axis=1, keepdims=True)
        shift = jnp.maximum(am, 0.0)
        p_mat = jnp.where(mask, acnt * jnp.exp(alpha - shift), 0.0)
        h2e = jnp.concatenate([h2, ones_col], axis=1)
        num_aug = jnp.concatenate(
            [jnp.dot(p_mat[t * M:(t + 1) * M, :],
                     h2e[t * M:(t + 1) * M, :],
                     preferred_element_type=f32) for t in range(G)], axis=0)
        num = num_aug[:, :fout]
        den = num_aug[:, fout:fout + 1]
        inv = 1.0 / (den + 1e-16)
        return jnp.maximum(num * inv + b_row[...], 0.0)

    def pool(h, fdim):
        hm = jnp.where(valid, h, -3e38)
        return jnp.max(hm.reshape(G, M, fdim), axis=1)

    x = x_ref[...]
    p0 = pool(x, 1)
    h1 = gat_layer(x, w1, b1, 8, True)
    h1_ref[...] = h1
    p1 = pool(h1, 8)
    h2 = gat_layer(h1, w2, b2, 64, False)
    h2_ref[...] = h2
    p2 = pool(h2, 64)
    h3 = gat_layer(h2, w3, b3, 32, False)
    h3_ref[...] = h3
    p3 = pool(h3, 32)
    h4 = gat_layer(h3, w4, b4, 9, False)
    h4_ref[...] = h4
    p4 = pool(h4, 9)
    pcat_ref[...] = jnp.concatenate([p0, p1, p2, p3, p4], axis=1)


def _run_gat(xp, a2, weights):
    (w1, b1, w2, b2, w3, b3, w4, b4) = weights
    nb = xp.shape[0] // M
    f32 = jnp.float32
    data_spec = lambda cols: pl.BlockSpec((ROWS, cols), lambda i: (i, 0))
    w_spec = lambda r, c: pl.BlockSpec((r, c), lambda i: (0, 0))
    pool_spec = lambda cols: pl.BlockSpec((G, cols), lambda i: (i, 0))
    out_shapes = [
        jax.ShapeDtypeStruct((nb * M, 8), f32),
        jax.ShapeDtypeStruct((nb * M, 64), f32),
        jax.ShapeDtypeStruct((nb * M, 32), f32),
        jax.ShapeDtypeStruct((nb * M, 9), f32),
        jax.ShapeDtypeStruct((nb, 114), f32),
    ]
    in_specs = [
        data_spec(1), data_spec(M),
        w_spec(1, 10), w_spec(1, 8),
        w_spec(8, 66), w_spec(1, 64),
        w_spec(64, 34), w_spec(1, 32),
        w_spec(32, 11), w_spec(1, 9),
    ]
    out_specs = [
        data_spec(8), data_spec(64), data_spec(32), data_spec(9),
        pool_spec(114),
    ]
    return pl.pallas_call(
        _gat_block_kernel,
        grid=(nb // G,),
        in_specs=in_specs,
        out_specs=out_specs,
        out_shape=out_shapes,
        compiler_params=pltpu.CompilerParams(dimension_semantics=("parallel",)),
    )(xp, a2, w1, b1, w2, b2, w3, b3, w4, b4)


def _mlp_kernel(res_ref, r1_ref, r2_ref, r3_ref, r4_ref, pc_ref,
                wr, wr1, wr2, wr3, wr4, wpc, bl1, wl2, bl2, wl3, bl3, o_ref):
    f32 = jnp.float32
    t = (jnp.dot(res_ref[...], wr[...], preferred_element_type=f32)
         + jnp.dot(r1_ref[...], wr1[...], preferred_element_type=f32)
         + jnp.dot(r2_ref[...], wr2[...], preferred_element_type=f32)
         + jnp.dot(r3_ref[...], wr3[...], preferred_element_type=f32)
         + jnp.dot(r4_ref[...], wr4[...], preferred_element_type=f32)
         + jnp.dot(pc_ref[...], wpc[...], preferred_element_type=f32)
         + bl1[...])
    t = jnp.maximum(t, 0.0)
    t = jnp.dot(t, wl2[...], preferred_element_type=f32) + bl2[...]
    t = jnp.maximum(t, 0.0)
    o_ref[...] = jnp.dot(t, wl3[...], preferred_element_type=f32) + bl3[...]


def _run_mlp(pieces, wl1_parts, bl1, wl2, bl2, wl3, bl3):
    f32 = jnp.float32
    nb = pieces[0].shape[0]
    w_spec = lambda r, c: pl.BlockSpec((r, c), lambda i: (0, 0))
    d_spec = lambda cols: pl.BlockSpec((MLP_ROWS, cols), lambda i: (i, 0))
    piece_cols = [NPG, NPG * 8, NPG * 64, NPG * 32, NPG * 9, 114]
    return pl.pallas_call(
        _mlp_kernel,
        grid=(nb // MLP_ROWS,),
        in_specs=(
            [d_spec(c) for c in piece_cols]
            + [w_spec(c, 1024) for c in piece_cols]
            + [w_spec(1, 1024), w_spec(1024, 128), w_spec(1, 128),
               w_spec(128, 9), w_spec(1, 9)]
        ),
        out_specs=pl.BlockSpec((MLP_ROWS, 9), lambda i: (i, 0)),
        out_shape=jax.ShapeDtypeStruct((nb, 9), f32),
        compiler_params=pltpu.CompilerParams(dimension_semantics=("parallel",)),
    )(*pieces, *wl1_parts, bl1, wl2, bl2, wl3, bl3)


def kernel(x, edge_index, batch, W1, as1, ad1, b1, W2, as2, ad2, b2,
           W3, as3, ad3, b3, W4, as4, ad4, b4, Wl1, bl1, Wl2, bl2, Wl3, bl3):
    del batch
    f32 = jnp.float32
    src = edge_index[0]
    dst = edge_index[1]

    row = lambda v: v.reshape(1, -1).astype(f32)
    aug = lambda w, a_s, a_d: jnp.concatenate(
        [w, (w @ a_s)[:, None], (w @ a_d)[:, None]], axis=1)
    weights = (aug(W1, as1, ad1), row(b1), aug(W2, as2, ad2), row(b2),
               aug(W3, as3, ad3), row(b3), aug(W4, as4, ad4), row(b4))
    offs = [0, 39, 351, 2847, 4095, 4446, 4560]
    wl1_parts = [Wl1[offs[i]:offs[i + 1]] for i in range(6)]
    bl = (row(bl1), Wl2, row(bl2), Wl3, row(bl3))

    a2 = _build_a(src, dst, B, 0).reshape(B * M, M)
    xp = jnp.pad(x.reshape(B, NPG), ((0, 0), (0, M - NPG))).reshape(B * M, 1)
    h1, h2, h3, h4, pcat = _run_gat(xp, a2, weights)

    res = x.reshape(B, NPG)
    res1 = h1.reshape(B, M, 8)[:, :NPG, :].reshape(B, NPG * 8)
    res2 = h2.reshape(B, M, 64)[:, :NPG, :].reshape(B, NPG * 64)
    res3 = h3.reshape(B, M, 32)[:, :NPG, :].reshape(B, NPG * 32)
    res4 = h4.reshape(B, M, 9)[:, :NPG, :].reshape(B, NPG * 9)
    pieces = (res, res1, res2, res3, res4, pcat)
    return _run_mlp(pieces, wl1_parts, *bl)


# speedup vs baseline: 1.0660x; 1.0001x over previous
import functools

import jax
import jax.numpy as jnp
from jax import lax
from jax.experimental import pallas as pl
from jax.experimental.pallas import tpu as pltpu
from jax.experimental.pallas import tpu_sc as plsc

B = 2048
NPG = 39
M = 40
DEG = 16
EPG = NPG * DEG
GM2 = M * M
NW = 32
GPW = B // NW
GPR = 16
ROUNDS = GPW // GPR

G = 64
ROWS = G * M
STEPS = B // G

MLP_ROWS = 512
F_DIM = 4560


def _make_sc_body(nb, hoff):
    gpw = nb // NW

    def _sc_body(src_hbm, dst_hbm, out_hbm, acc, sbuf, dbuf):
        c = lax.axis_index("c")
        s = lax.axis_index("s")
        wid = s * 2 + c
        lane = lax.iota(jnp.int32, 16)
        ones = jnp.ones((16,), jnp.float32)
        zeros = jnp.zeros((16,), jnp.float32)

        def round_body(r, carry):
            gl = wid * gpw + r * GPR

            def zero_body(i, carry2):
                for u in range(8):
                    acc[pl.ds((i * 8 + u) * 16, 16)] = zeros
                return carry2

            lax.fori_loop(0, GPR * GM2 // 128, zero_body, 0)

            pltpu.sync_copy(src_hbm.at[pl.ds(gl * EPG, GPR * EPG)], sbuf)
            pltpu.sync_copy(dst_hbm.at[pl.ds(gl * EPG, GPR * EPG)], dbuf)

            gbase = (hoff + gl + lane) * NPG
            abase = lane * GM2

            def edge_body(j, carry2):
                for u in range(8):
                    ei = lane * EPG + (j * 8 + u)
                    sv = plsc.load_gather(sbuf, [ei])
                    dv = plsc.load_gather(dbuf, [ei])
                    idx = abase + (dv - gbase) * M + (sv - gbase)
                    plsc.addupdate_scatter(acc, [idx], ones)
                return carry2

            lax.fori_loop(0, EPG // 8, edge_body, 0)

            pltpu.sync_copy(acc, out_hbm.at[pl.ds(gl * GM2, GPR * GM2)])
            return carry

        lax.fori_loop(0, gpw // GPR, round_body, 0)

    return _sc_body


def _build_a(src, dst, nb, hoff):
    mesh = plsc.VectorSubcoreMesh(core_axis_name="c", subcore_axis_name="s")
    call = functools.partial(
        pl.kernel,
        mesh=mesh,
        compiler_params=pltpu.CompilerParams(needs_layout_passes=False),
        out_type=jax.ShapeDtypeStruct((nb * GM2,), jnp.float32),
        scratch_types=[
            pltpu.VMEM((GPR * GM2,), jnp.float32),
            pltpu.VMEM((GPR * EPG,), jnp.int32),
            pltpu.VMEM((GPR * EPG,), jnp.int32),
        ],
    )(_make_sc_body(nb, hoff))
    return call(src, dst)


def _gat_block_kernel(x_ref, a_ref,
                      w1, b1, w2, b2, w3, b3, w4, b4,
                      h1_ref, h2_ref, h3_ref, h4_ref, pcat_ref):
    f32 = jnp.float32
    rowl = lax.broadcasted_iota(jnp.int32, (ROWS, 1), 0) % M
    valid = rowl < NPG
    lane_id = lax.broadcasted_iota(jnp.int32, (ROWS, M), 1)
    patt = (lane_id == rowl).astype(f32)
    self_loop = jnp.where((lane_id == rowl) & valid, 1.0, 0.0)
    acnt = a_ref[...] + self_loop
    mask = acnt > 0

    sm = (lax.broadcasted_iota(jnp.int32, (G, ROWS), 1) // M
          == lax.broadcasted_iota(jnp.int32, (G, ROWS), 0)).astype(f32)
    rm = (lax.broadcasted_iota(jnp.int32, (ROWS, G), 0) // M
          == lax.broadcasted_iota(jnp.int32, (ROWS, G), 1)).astype(f32)
    ones_col = jnp.ones((ROWS, 1), f32)
    ones_row = jnp.ones((1, M), f32)

    def gat_layer(h_in, waug, b_row, fout, first):
        if first:
            h2aug = h_in * waug[...]
        else:
            h2aug = jnp.dot(h_in, waug[...], preferred_element_type=f32)
        h2 = h2aug[:, :fout]
        asrc = h2aug[:, fout:fout + 1]
        adst = h2aug[:, fout + 1:fout + 2]
        rhs = asrc * patt
        grid = jnp.dot(sm, rhs, preferred_element_type=f32)
        r_mat = jnp.dot(rm, grid, preferred_element_type=f32)
        pre = r_mat + adst
        alpha = jnp.where(pre >= 0, pre, 0.2 * pre)
        am = jnp.max(alpha, axis=1, keepdims=True)
        shift = jnp.maximum(am, 0.0)
        p_mat = jnp.where(mask, acnt * jnp.exp(alpha - shift), 0.0)
        h2e = jnp.concatenate([h2, ones_col], axis=1)
        num_aug = jnp.concatenate(
            [jnp.dot(p_mat[t * M:(t + 1) * M, :],
                     h2e[t * M:(t + 1) * M, :],
                     preferred_element_type=f32) for t in range(G)], axis=0)
        num = num_aug[:, :fout]
        den = num_aug[:, fout:fout + 1]
        inv = 1.0 / (den + 1e-16)
        return jnp.maximum(num * inv + b_row[...], 0.0)

    def pool(h, fdim):
        hm = jnp.where(valid, h, -3e38)
        return jnp.max(hm.reshape(G, M, fdim), axis=1)

    x = x_ref[...]
    p0 = pool(x, 1)
    h1 = gat_layer(x, w1, b1, 8, True)
    h1_ref[...] = h1
    p1 = pool(h1, 8)
    h2 = gat_layer(h1, w2, b2, 64, False)
    h2_ref[...] = h2
    p2 = pool(h2, 64)
    h3 = gat_layer(h2, w3, b3, 32, False)
    h3_ref[...] = h3
    p3 = pool(h3, 32)
    h4 = gat_layer(h3, w4, b4, 9, False)
    h4_ref[...] = h4
    p4 = pool(h4, 9)
    pcat_ref[...] = jnp.concatenate([p0, p1, p2, p3, p4], axis=1)


def _run_gat(xp, a2, weights):
    (w1, b1, w2, b2, w3, b3, w4, b4) = weights
    nb = xp.shape[0] // M
    f32 = jnp.float32
    data_spec = lambda cols: pl.BlockSpec((ROWS, cols), lambda i: (i, 0))
    w_spec = lambda r, c: pl.BlockSpec((r, c), lambda i: (0, 0))
    pool_spec = lambda cols: pl.BlockSpec((G, cols), lambda i: (i, 0))
    out_shapes = [
        jax.ShapeDtypeStruct((nb * M, 8), f32),
        jax.ShapeDtypeStruct((nb * M, 64), f32),
        jax.ShapeDtypeStruct((nb * M, 32), f32),
        jax.ShapeDtypeStruct((nb * M, 9), f32),
        jax.ShapeDtypeStruct((nb, 114), f32),
    ]
    in_specs = [
        data_spec(1), data_spec(M),
        w_spec(1, 10), w_spec(1, 8),
        w_spec(8, 66), w_spec(1, 64),
        w_spec(64, 34), w_spec(1, 32),
        w_spec(32, 11), w_spec(1, 9),
    ]
    out_specs = [
        data_spec(8), data_spec(64), data_spec(32), data_spec(9),
        pool_spec(114),
    ]
    return pl.pallas_call(
        _gat_block_kernel,
        grid=(nb // G,),
        in_specs=in_specs,
        out_specs=out_specs,
        out_shape=out_shapes,
        compiler_params=pltpu.CompilerParams(dimension_semantics=("parallel",)),
    )(xp, a2, w1, b1, w2, b2, w3, b3, w4, b4)


def _mlp_kernel(res_ref, r1_ref, r2_ref, r3_ref, r4_ref, pc_ref,
                wr, wr1, wr2, wr3, wr4, wpc, bl1, wl2, bl2, wl3, bl3, o_ref):
    f32 = jnp.float32
    t = (jnp.dot(res_ref[...], wr[...], preferred_element_type=f32)
         + jnp.dot(r1_ref[...], wr1[...], preferred_element_type=f32)
         + jnp.dot(r2_ref[...], wr2[...], preferred_element_type=f32)
         + jnp.dot(r3_ref[...], wr3[...], preferred_element_type=f32)
         + jnp.dot(r4_ref[...], wr4[...], preferred_element_type=f32)
         + jnp.dot(pc_ref[...], wpc[...], preferred_element_type=f32)
         + bl1[...])
    t = jnp.maximum(t, 0.0)
    t = jnp.dot(t, wl2[...], preferred_element_type=f32) + bl2[...]
    t = jnp.maximum(t, 0.0)
    o_ref[...] = jnp.dot(t, wl3[...], preferred_element_type=f32) + bl3[...]


def _run_mlp(pieces, wl1_parts, bl1, wl2, bl2, wl3, bl3):
    f32 = jnp.float32
    nb = pieces[0].shape[0]
    w_spec = lambda r, c: pl.BlockSpec((r, c), lambda i: (0, 0))
    d_spec = lambda cols: pl.BlockSpec((MLP_ROWS, cols), lambda i: (i, 0))
    piece_cols = [NPG, NPG * 8, NPG * 64, NPG * 32, NPG * 9, 114]
    return pl.pallas_call(
        _mlp_kernel,
        grid=(nb // MLP_ROWS,),
        in_specs=(
            [d_spec(c) for c in piece_cols]
            + [w_spec(c, 1024) for c in piece_cols]
            + [w_spec(1, 1024), w_spec(1024, 128), w_spec(1, 128),
               w_spec(128, 9), w_spec(1, 9)]
        ),
        out_specs=pl.BlockSpec((MLP_ROWS, 9), lambda i: (i, 0)),
        out_shape=jax.ShapeDtypeStruct((nb, 9), f32),
        compiler_params=pltpu.CompilerParams(dimension_semantics=("parallel",)),
    )(*pieces, *wl1_parts, bl1, wl2, bl2, wl3, bl3)


def kernel(x, edge_index, batch, W1, as1, ad1, b1, W2, as2, ad2, b2,
           W3, as3, ad3, b3, W4, as4, ad4, b4, Wl1, bl1, Wl2, bl2, Wl3, bl3):
    del batch
    f32 = jnp.float32
    src = edge_index[0]
    dst = edge_index[1]

    row = lambda v: v.reshape(1, -1).astype(f32)
    aug = lambda w, a_s, a_d: jnp.concatenate(
        [w, (w @ a_s)[:, None], (w @ a_d)[:, None]], axis=1)
    weights = (aug(W1, as1, ad1), row(b1), aug(W2, as2, ad2), row(b2),
               aug(W3, as3, ad3), row(b3), aug(W4, as4, ad4), row(b4))
    offs = [0, 39, 351, 2847, 4095, 4446, 4560]
    wl1_parts = [Wl1[offs[i]:offs[i + 1]] for i in range(6)]
    bl = (row(bl1), Wl2, row(bl2), Wl3, row(bl3))

    a2 = _build_a(src, dst, B, 0).reshape(B * M, M)
    xp = jnp.pad(x.reshape(B, NPG), ((0, 0), (0, M - NPG))).reshape(B * M, 1)
    h1, h2, h3, h4, pcat = _run_gat(xp, a2, weights)

    res = x.reshape(B, NPG)
    res1 = h1.reshape(B, M, 8)[:, :NPG, :].reshape(B, NPG * 8)
    res2 = h2.reshape(B, M, 64)[:, :NPG, :].reshape(B, NPG * 64)
    res3 = h3.reshape(B, M, 32)[:, :NPG, :].reshape(B, NPG * 32)
    res4 = h4.reshape(B, M, 9)[:, :NPG, :].reshape(B, NPG * 9)
    pieces = (res, res1, res2, res3, res4, pcat)
    return _run_mlp(pieces, wl1_parts, *bl)
